# baseline (speedup 1.0000x reference)
import functools

import jax
import jax.numpy as jnp
from jax import lax
from jax.experimental import pallas as pl
from jax.experimental.pallas import tpu as pltpu
from jax.experimental.pallas import tpu_sc as plsc

_N = 10000
_NPAD = 10240
_F = 128
_H = 128
_C = 40
_CP = 48
_E = 320000
_NC = 2
_NS = 16
_NW = _NC * _NS
_EPW = _E // _NW
_CH = 80
_NCHUNK = _EPW // _CH
_WIN = 25
_NWIN = _NCHUNK // _WIN
_RPS = _NPAD // _NS
_RB = 1000

_mesh = plsc.VectorSubcoreMesh(core_axis_name="c", subcore_axis_name="s")


@functools.partial(
    pl.kernel,
    out_type=jax.ShapeDtypeStruct((_NC, _NPAD), jnp.float32),
    mesh=_mesh,
    scratch_types=[
        pltpu.VMEM_SHARED((_NPAD,), jnp.float32),
        pltpu.VMEM((_NWIN, _WIN, _CH), jnp.int32),
        pltpu.VMEM((_NWIN, _WIN, _CH), jnp.float32),
        pltpu.VMEM((_RPS,), jnp.float32),
    ],
)
def _deg_kernel(dst_hbm, w_hbm, out_hbm, acc, di, wi, zbuf):
  c = lax.axis_index("c")
  s = lax.axis_index("s")
  wid = c * _NS + s

  @pl.loop(0, _RPS, step=16)
  def _(r):
    zbuf[pl.ds(r, 16)] = jnp.zeros((16,), jnp.float32)

  pltpu.sync_copy(zbuf, acc.at[pl.ds(s * _RPS, _RPS)])
  pltpu.sync_copy(dst_hbm.at[wid], di)
  pltpu.sync_copy(w_hbm.at[wid], wi)
  plsc.subcore_barrier()

  @pl.loop(0, _NWIN)
  def _(jw):
    @pl.loop(0, _WIN)
    def _(j):
      pltpu.sync_copy(wi.at[jw, j], acc.at[di.at[jw, j]], add=True)

  plsc.subcore_barrier()
  pltpu.sync_copy(acc.at[pl.ds(s * _RPS, _RPS)],
                  out_hbm.at[c, pl.ds(s * _RPS, _RPS)])


def _make_agg(D):
  @functools.partial(
      pl.kernel,
      out_type=jax.ShapeDtypeStruct((_NC, _NPAD, D), jnp.float32),
      mesh=_mesh,
      scratch_types=[
          pltpu.VMEM_SHARED((_NPAD, D), jnp.float32),
          pltpu.VMEM((_WIN, _CH), jnp.int32),
          pltpu.VMEM((_WIN, _CH), jnp.int32),
          pltpu.VMEM((_WIN, _CH), jnp.float32),
          pltpu.VMEM((3, _CH, D), jnp.float32),
          pltpu.SemaphoreType.DMA((3,)),
          pltpu.SemaphoreType.DMA((3,)),
      ],
      compiler_params=pltpu.CompilerParams(use_tc_tiling_on_sc=False),
  )
  def agg(h_hbm, src_hbm, dst_hbm, w_hbm, out_hbm, acc, si, di, wi, rows,
          gsem, ssem):
    c = lax.axis_index("c")
    s = lax.axis_index("s")
    wid = c * _NS + s

    @pl.loop(0, _CH)
    def _(r):
      for d in range(D // 16):
        rows[0, r, pl.ds(d * 16, 16)] = jnp.zeros((16,), jnp.float32)

    @pl.loop(0, _RPS, step=_CH)
    def _(r0):
      pltpu.sync_copy(rows.at[0], acc.at[pl.ds(s * _RPS + r0, _CH)])

    plsc.subcore_barrier()

    @pl.loop(0, _NWIN)
    def _(jw):
      pltpu.sync_copy(src_hbm.at[wid, jw], si)
      pltpu.sync_copy(dst_hbm.at[wid, jw], di)
      pltpu.sync_copy(w_hbm.at[wid, jw], wi)

      pltpu.async_copy(h_hbm.at[si.at[0]], rows.at[0], gsem.at[0])
      pltpu.async_copy(h_hbm.at[si.at[1]], rows.at[1], gsem.at[1])

      @pl.loop(0, _WIN)
      def _(j):
        slot = lax.rem(j, 3)
        pltpu.make_async_copy(h_hbm.at[si.at[j]], rows.at[slot],
                              gsem.at[slot]).wait()

        @pl.loop(0, _CH, step=16)
        def _(g):
          wvec = wi[j, pl.ds(g, 16)]
          for l in range(16):
            wv = wvec[l]
            for d in range(D // 16):
              rows[slot, g + l, pl.ds(d * 16, 16)] = (
                  rows[slot, g + l, pl.ds(d * 16, 16)] * wv)

        pltpu.async_copy(rows.at[slot], acc.at[di.at[j]], ssem.at[slot],
                         add=True)

        @pl.when(j + 2 < _WIN)
        def _():
          nslot = lax.rem(j + 2, 3)

          @pl.when(j >= 1)
          def _():
            pltpu.make_async_copy(rows.at[nslot], acc.at[di.at[j - 1]],
                                  ssem.at[nslot]).wait()

          pltpu.async_copy(h_hbm.at[si.at[j + 2]], rows.at[nslot],
                           gsem.at[nslot])

      for jt in (_WIN - 3, _WIN - 2, _WIN - 1):
        pltpu.make_async_copy(rows.at[jt % 3], acc.at[di.at[jt]],
                              ssem.at[jt % 3]).wait()

    plsc.subcore_barrier()
    pltpu.sync_copy(acc.at[pl.ds(s * _RPS, _RPS)],
                    out_hbm.at[c, pl.ds(s * _RPS, _RPS)])

  return agg


_agg128 = _make_agg(_H)
_agg48 = _make_agg(_CP)


def _tc_a_body(degp_ref, x_ref, w1_ref, hp_ref, dinv_ref):
  deg = degp_ref[:, 0:1] + degp_ref[:, 1:2] + 1.0
  dinv = lax.rsqrt(deg)
  p = jnp.dot(x_ref[...], w1_ref[...], preferred_element_type=jnp.float32)
  hp_ref[...] = p * dinv
  dinv_ref[...] = dinv


def _tc_a(degp_t, x, W1):
  return pl.pallas_call(
      _tc_a_body,
      grid=(_N // _RB,),
      in_specs=[
          pl.BlockSpec((_RB, _NC), lambda i: (i, 0)),
          pl.BlockSpec((_RB, _F), lambda i: (i, 0)),
          pl.BlockSpec((_F, _H), lambda i: (0, 0)),
      ],
      out_specs=[
          pl.BlockSpec((_RB, _H), lambda i: (i, 0)),
          pl.BlockSpec((_RB, 1), lambda i: (i, 0)),
      ],
      out_shape=[
          jax.ShapeDtypeStruct((_N, _H), jnp.float32),
          jax.ShapeDtypeStruct((_N, 1), jnp.float32),
      ],
  )(degp_t, x, W1)


def _tc_b_body(agg_ref, hp_ref, dinv_ref, b1_ref, w2_ref, h2p_ref):
  ssum = agg_ref[0] + agg_ref[1] + hp_ref[...]
  o1 = jnp.maximum(ssum * dinv_ref[...] + b1_ref[...], 0.0)
  p2 = jnp.dot(o1, w2_ref[...], preferred_element_type=jnp.float32)
  h2p = p2 * dinv_ref[...]
  h2p_ref[...] = jnp.concatenate(
      [h2p, jnp.zeros((_RB, _CP - _C), jnp.float32)], axis=1)


def _tc_b(agg1, hp, dinv, b1, W2):
  return pl.pallas_call(
      _tc_b_body,
      grid=(_N // _RB,),
      in_specs=[
          pl.BlockSpec((_NC, _RB, _H), lambda i: (0, i, 0)),
          pl.BlockSpec((_RB, _H), lambda i: (i, 0)),
          pl.BlockSpec((_RB, 1), lambda i: (i, 0)),
          pl.BlockSpec((1, _H), lambda i: (0, 0)),
          pl.BlockSpec((_H, _C), lambda i: (0, 0)),
      ],
      out_specs=pl.BlockSpec((_RB, _CP), lambda i: (i, 0)),
      out_shape=jax.ShapeDtypeStruct((_N, _CP), jnp.float32),
  )(agg1, hp, dinv, b1, W2)


def _tc_c_body(agg_ref, h2p_ref, dinv_ref, b2_ref, out_ref):
  ssum = (agg_ref[0] + agg_ref[1] + h2p_ref[...]) * dinv_ref[...]
  v = ssum[:, :_C] + b2_ref[...]
  m = jnp.max(v, axis=1, keepdims=True)
  lse = jnp.log(jnp.sum(jnp.exp(v - m), axis=1, keepdims=True)) + m
  out_ref[...] = v - lse


def _tc_c(agg2, h2p, dinv, b2):
  return pl.pallas_call(
      _tc_c_body,
      grid=(_N // _RB,),
      in_specs=[
          pl.BlockSpec((_NC, _RB, _CP), lambda i: (0, i, 0)),
          pl.BlockSpec((_RB, _CP), lambda i: (i, 0)),
          pl.BlockSpec((_RB, 1), lambda i: (i, 0)),
          pl.BlockSpec((1, _C), lambda i: (0, 0)),
      ],
      out_specs=pl.BlockSpec((_RB, _C), lambda i: (i, 0)),
      out_shape=jax.ShapeDtypeStruct((_N, _C), jnp.float32),
  )(agg2, h2p, dinv, b2)


def kernel(x, edge_index, edge_weight, W1, b1, W2, b2):
  src = edge_index[0].reshape(_NW, _NWIN, _WIN, _CH)
  dst = edge_index[1].reshape(_NW, _NWIN, _WIN, _CH)
  w4 = edge_weight.reshape(_NW, _NWIN, _WIN, _CH)

  degp = _deg_kernel(dst, w4)
  degp_t = degp.T[:_N]
  hp, dinv = _tc_a(degp_t, x, W1)
  agg1 = _agg128(hp, src, dst, w4)
  h2p = _tc_b(agg1, hp, dinv, b1.reshape(1, _H), W2)
  agg2 = _agg48(h2p, src, dst, w4)
  return _tc_c(agg2, h2p, dinv, b2.reshape(1, _C))


# speedup vs baseline: 2.0305x; 2.0305x over previous
import functools

import jax
import jax.numpy as jnp
from jax import lax
from jax.experimental import pallas as pl
from jax.experimental.pallas import tpu as pltpu
from jax.experimental.pallas import tpu_sc as plsc

_N = 10000
_NPAD = 10240
_F = 128
_H = 128
_C = 40
_CP = 48
_E = 320000
_NC = 2
_NS = 16
_NW = _NC * _NS
_EPW = _E // _NW
_CH = 80
_NCHUNK = _EPW // _CH
_WIN = 25
_NWIN = _NCHUNK // _WIN
_RPS = _NPAD // _NS
_RB = 1000

_mesh = plsc.VectorSubcoreMesh(core_axis_name="c", subcore_axis_name="s")


@functools.partial(
    pl.kernel,
    out_type=jax.ShapeDtypeStruct((_NC, _NPAD), jnp.float32),
    mesh=_mesh,
    scratch_types=[
        pltpu.VMEM_SHARED((_NPAD,), jnp.float32),
        pltpu.VMEM((_NWIN, _WIN, _CH), jnp.int32),
        pltpu.VMEM((_NWIN, _WIN, _CH), jnp.float32),
        pltpu.VMEM((_RPS,), jnp.float32),
    ],
)
def _deg_kernel(dst_hbm, w_hbm, out_hbm, acc, di, wi, zbuf):
  c = lax.axis_index("c")
  s = lax.axis_index("s")
  wid = c * _NS + s

  @pl.loop(0, _RPS, step=16)
  def _(r):
    zbuf[pl.ds(r, 16)] = jnp.zeros((16,), jnp.float32)

  pltpu.sync_copy(zbuf, acc.at[pl.ds(s * _RPS, _RPS)])
  pltpu.sync_copy(dst_hbm.at[wid], di)
  pltpu.sync_copy(w_hbm.at[wid], wi)
  plsc.subcore_barrier()

  @pl.loop(0, _NWIN)
  def _(jw):
    @pl.loop(0, _WIN)
    def _(j):
      pltpu.sync_copy(wi.at[jw, j], acc.at[di.at[jw, j]], add=True)

  plsc.subcore_barrier()
  pltpu.sync_copy(acc.at[pl.ds(s * _RPS, _RPS)],
                  out_hbm.at[c, pl.ds(s * _RPS, _RPS)])


def _make_agg(D):
  @functools.partial(
      pl.kernel,
      out_type=jax.ShapeDtypeStruct((_NC, _NPAD, D), jnp.float32),
      mesh=_mesh,
      scratch_types=[
          pltpu.VMEM_SHARED((_NPAD, D), jnp.float32),
          pltpu.VMEM((_WIN, _CH), jnp.int32),
          pltpu.VMEM((_WIN, _CH), jnp.int32),
          pltpu.VMEM((_WIN, _CH), jnp.float32),
          pltpu.VMEM((3, _CH, D), jnp.float32),
          pltpu.SemaphoreType.DMA((3,)),
          pltpu.SemaphoreType.DMA((3,)),
      ],
      compiler_params=pltpu.CompilerParams(use_tc_tiling_on_sc=False),
  )
  def agg(h_hbm, src_hbm, dst_hbm, w_hbm, out_hbm, acc, si, di, wi, rows,
          gsem, ssem):
    c = lax.axis_index("c")
    s = lax.axis_index("s")
    wid = c * _NS + s

    @pl.loop(0, _CH)
    def _(r):
      for d in range(D // 16):
        rows[0, r, pl.ds(d * 16, 16)] = jnp.zeros((16,), jnp.float32)

    @pl.loop(0, _RPS, step=_CH)
    def _(r0):
      pltpu.sync_copy(rows.at[0], acc.at[pl.ds(s * _RPS + r0, _CH)])

    plsc.subcore_barrier()

    @pl.loop(0, _NWIN)
    def _(jw):
      pltpu.sync_copy(src_hbm.at[wid, jw], si)
      pltpu.sync_copy(dst_hbm.at[wid, jw], di)
      pltpu.sync_copy(w_hbm.at[wid, jw], wi)

      def mul_rows(p, j):
        @pl.loop(0, _CH, step=16)
        def _(g):
          wvec = wi[j, pl.ds(g, 16)]
          for l in range(16):
            wv = wvec[l]
            for d in range(D // 16):
              rows[p, g + l, pl.ds(d * 16, 16)] = (
                  rows[p, g + l, pl.ds(d * 16, 16)] * wv)

      def wait_gather(p, j):
        pltpu.make_async_copy(h_hbm.at[si.at[j]], rows.at[p],
                              gsem.at[p]).wait()

      def issue_gather(p, j):
        pltpu.async_copy(h_hbm.at[si.at[j]], rows.at[p], gsem.at[p])

      def issue_scatter(p, j):
        pltpu.async_copy(rows.at[p], acc.at[di.at[j]], ssem.at[p], add=True)

      def wait_scatter(p, j):
        pltpu.make_async_copy(rows.at[p], acc.at[di.at[j]],
                              ssem.at[p]).wait()

      issue_gather(0, 0)
      issue_gather(1, 1)

      @pl.loop(0, _WIN - 1, step=3)
      def _(j0):
        wait_gather(0, j0)
        mul_rows(0, j0)
        issue_scatter(0, j0)

        @pl.when(j0 > 0)
        def _():
          wait_scatter(2, j0 - 1)
        issue_gather(2, j0 + 2)

        wait_gather(1, j0 + 1)
        mul_rows(1, j0 + 1)
        issue_scatter(1, j0 + 1)

        wait_scatter(0, j0)
        issue_gather(0, j0 + 3)

        wait_gather(2, j0 + 2)
        mul_rows(2, j0 + 2)
        issue_scatter(2, j0 + 2)

        wait_scatter(1, j0 + 1)

        @pl.when(j0 < _WIN - 4)
        def _():
          issue_gather(1, j0 + 4)

      wait_gather(0, _WIN - 1)
      mul_rows(0, _WIN - 1)
      issue_scatter(0, _WIN - 1)
      wait_scatter(2, _WIN - 2)
      wait_scatter(0, _WIN - 1)

    plsc.subcore_barrier()
    pltpu.sync_copy(acc.at[pl.ds(s * _RPS, _RPS)],
                    out_hbm.at[c, pl.ds(s * _RPS, _RPS)])

  return agg


_agg128 = _make_agg(_H)
_agg48 = _make_agg(_CP)


def _tc_a_body(degp_ref, x_ref, w1_ref, hp_ref, dinv_ref):
  deg = degp_ref[:, 0:1] + degp_ref[:, 1:2] + 1.0
  dinv = lax.rsqrt(deg)
  p = jnp.dot(x_ref[...], w1_ref[...], preferred_element_type=jnp.float32)
  hp_ref[...] = p * dinv
  dinv_ref[...] = dinv


def _tc_a(degp_t, x, W1):
  return pl.pallas_call(
      _tc_a_body,
      grid=(_N // _RB,),
      in_specs=[
          pl.BlockSpec((_RB, _NC), lambda i: (i, 0)),
          pl.BlockSpec((_RB, _F), lambda i: (i, 0)),
          pl.BlockSpec((_F, _H), lambda i: (0, 0)),
      ],
      out_specs=[
          pl.BlockSpec((_RB, _H), lambda i: (i, 0)),
          pl.BlockSpec((_RB, 1), lambda i: (i, 0)),
      ],
      out_shape=[
          jax.ShapeDtypeStruct((_N, _H), jnp.float32),
          jax.ShapeDtypeStruct((_N, 1), jnp.float32),
      ],
  )(degp_t, x, W1)


def _tc_b_body(agg_ref, hp_ref, dinv_ref, b1_ref, w2_ref, h2p_ref):
  ssum = agg_ref[0] + agg_ref[1] + hp_ref[...]
  o1 = jnp.maximum(ssum * dinv_ref[...] + b1_ref[...], 0.0)
  p2 = jnp.dot(o1, w2_ref[...], preferred_element_type=jnp.float32)
  h2p = p2 * dinv_ref[...]
  h2p_ref[...] = jnp.concatenate(
      [h2p, jnp.zeros((_RB, _CP - _C), jnp.float32)], axis=1)


def _tc_b(agg1, hp, dinv, b1, W2):
  return pl.pallas_call(
      _tc_b_body,
      grid=(_N // _RB,),
      in_specs=[
          pl.BlockSpec((_NC, _RB, _H), lambda i: (0, i, 0)),
          pl.BlockSpec((_RB, _H), lambda i: (i, 0)),
          pl.BlockSpec((_RB, 1), lambda i: (i, 0)),
          pl.BlockSpec((1, _H), lambda i: (0, 0)),
          pl.BlockSpec((_H, _C), lambda i: (0, 0)),
      ],
      out_specs=pl.BlockSpec((_RB, _CP), lambda i: (i, 0)),
      out_shape=jax.ShapeDtypeStruct((_N, _CP), jnp.float32),
  )(agg1, hp, dinv, b1, W2)


def _tc_c_body(agg_ref, h2p_ref, dinv_ref, b2_ref, out_ref):
  ssum = (agg_ref[0] + agg_ref[1] + h2p_ref[...]) * dinv_ref[...]
  v = ssum[:, :_C] + b2_ref[...]
  m = jnp.max(v, axis=1, keepdims=True)
  lse = jnp.log(jnp.sum(jnp.exp(v - m), axis=1, keepdims=True)) + m
  out_ref[...] = v - lse


def _tc_c(agg2, h2p, dinv, b2):
  return pl.pallas_call(
      _tc_c_body,
      grid=(_N // _RB,),
      in_specs=[
          pl.BlockSpec((_NC, _RB, _CP), lambda i: (0, i, 0)),
          pl.BlockSpec((_RB, _CP), lambda i: (i, 0)),
          pl.BlockSpec((_RB, 1), lambda i: (i, 0)),
          pl.BlockSpec((1, _C), lambda i: (0, 0)),
      ],
      out_specs=pl.BlockSpec((_RB, _C), lambda i: (i, 0)),
      out_shape=jax.ShapeDtypeStruct((_N, _C), jnp.float32),
  )(agg2, h2p, dinv, b2)


def kernel(x, edge_index, edge_weight, W1, b1, W2, b2):
  src = edge_index[0].reshape(_NW, _NWIN, _WIN, _CH)
  dst = edge_index[1].reshape(_NW, _NWIN, _WIN, _CH)
  w4 = edge_weight.reshape(_NW, _NWIN, _WIN, _CH)

  degp = _deg_kernel(dst, w4)
  degp_t = degp.T[:_N]
  hp, dinv = _tc_a(degp_t, x, W1)
  agg1 = _agg128(hp, src, dst, w4)
  h2p = _tc_b(agg1, hp, dinv, b1.reshape(1, _H), W2)
  agg2 = _agg48(h2p, src, dst, w4)
  return _tc_c(agg2, h2p, dinv, b2.reshape(1, _C))


# speedup vs baseline: 2.0552x; 1.0122x over previous
import functools

import jax
import jax.numpy as jnp
from jax import lax
from jax.experimental import pallas as pl
from jax.experimental.pallas import tpu as pltpu
from jax.experimental.pallas import tpu_sc as plsc

_N = 10000
_NPAD = 10240
_F = 128
_H = 128
_C = 40
_CP = 48
_E = 320000
_NC = 2
_NS = 16
_NW = _NC * _NS
_EPW = _E // _NW
_CH = 80
_NCHUNK = _EPW // _CH
_WIN = 25
_NWIN = _NCHUNK // _WIN
_RPS = _NPAD // _NS
_RB = 1000

_mesh = plsc.VectorSubcoreMesh(core_axis_name="c", subcore_axis_name="s")


@functools.partial(
    pl.kernel,
    out_type=jax.ShapeDtypeStruct((_NC, _NPAD), jnp.float32),
    mesh=_mesh,
    scratch_types=[
        pltpu.VMEM_SHARED((_NPAD,), jnp.float32),
        pltpu.VMEM((_NWIN, _WIN, _CH), jnp.int32),
        pltpu.VMEM((_NWIN, _WIN, _CH), jnp.float32),
        pltpu.VMEM((_RPS,), jnp.float32),
    ],
    compiler_params=pltpu.CompilerParams(use_tc_tiling_on_sc=False),
)
def _deg_kernel(dst_hbm, w_hbm, out_hbm, acc, di, wi, zbuf):
  c = lax.axis_index("c")
  s = lax.axis_index("s")
  wid = c * _NS + s

  @pl.loop(0, _RPS, step=16)
  def _(r):
    zbuf[pl.ds(r, 16)] = jnp.zeros((16,), jnp.float32)

  pltpu.sync_copy(zbuf, acc.at[pl.ds(s * _RPS, _RPS)])
  pltpu.sync_copy(dst_hbm.at[wid], di)
  pltpu.sync_copy(w_hbm.at[wid], wi)
  plsc.subcore_barrier()

  @pl.loop(0, _NWIN)
  def _(jw):
    @pl.loop(0, _WIN)
    def _(j):
      pltpu.sync_copy(wi.at[jw, j], acc.at[di.at[jw, j]], add=True)

  plsc.subcore_barrier()
  pltpu.sync_copy(acc.at[pl.ds(s * _RPS, _RPS)],
                  out_hbm.at[c, pl.ds(s * _RPS, _RPS)])


def _make_agg(D):
  @functools.partial(
      pl.kernel,
      out_type=jax.ShapeDtypeStruct((_NC, _NPAD, D), jnp.float32),
      mesh=_mesh,
      scratch_types=[
          pltpu.VMEM_SHARED((_NPAD, D), jnp.float32),
          pltpu.VMEM((_WIN, _CH), jnp.int32),
          pltpu.VMEM((_WIN, _CH), jnp.int32),
          pltpu.VMEM((_WIN, _CH), jnp.float32),
          pltpu.VMEM((3, _CH, D), jnp.float32),
          pltpu.SemaphoreType.DMA((3,)),
          pltpu.SemaphoreType.DMA((3,)),
      ],
      compiler_params=pltpu.CompilerParams(use_tc_tiling_on_sc=False),
  )
  def agg(h_hbm, src_hbm, dst_hbm, w_hbm, out_hbm, acc, si, di, wi, rows,
          gsem, ssem):
    c = lax.axis_index("c")
    s = lax.axis_index("s")
    wid = c * _NS + s

    @pl.loop(0, _CH)
    def _(r):
      for d in range(D // 16):
        rows[0, r, pl.ds(d * 16, 16)] = jnp.zeros((16,), jnp.float32)

    @pl.loop(0, _RPS, step=_CH)
    def _(r0):
      pltpu.sync_copy(rows.at[0], acc.at[pl.ds(s * _RPS + r0, _CH)])

    plsc.subcore_barrier()

    @pl.loop(0, _NWIN)
    def _(jw):
      pltpu.sync_copy(src_hbm.at[wid, jw], si)
      pltpu.sync_copy(dst_hbm.at[wid, jw], di)
      pltpu.sync_copy(w_hbm.at[wid, jw], wi)

      def mul_rows(p, j):
        @pl.loop(0, _CH, step=16)
        def _(g):
          wvec = wi[j, pl.ds(g, 16)]
          for l in range(16):
            wv = wvec[l]
            for d in range(D // 16):
              rows[p, g + l, pl.ds(d * 16, 16)] = (
                  rows[p, g + l, pl.ds(d * 16, 16)] * wv)

      def wait_gather(p, j):
        pltpu.make_async_copy(h_hbm.at[si.at[j]], rows.at[p],
                              gsem.at[p]).wait()

      def issue_gather(p, j):
        pltpu.async_copy(h_hbm.at[si.at[j]], rows.at[p], gsem.at[p])

      def issue_scatter(p, j):
        pltpu.async_copy(rows.at[p], acc.at[di.at[j]], ssem.at[p], add=True)

      def wait_scatter(p, j):
        pltpu.make_async_copy(rows.at[p], acc.at[di.at[j]],
                              ssem.at[p]).wait()

      issue_gather(0, 0)
      issue_gather(1, 1)

      @pl.loop(0, _WIN - 1, step=3)
      def _(j0):
        wait_gather(0, j0)
        mul_rows(0, j0)
        issue_scatter(0, j0)

        @pl.when(j0 > 0)
        def _():
          wait_scatter(2, j0 - 1)
        issue_gather(2, j0 + 2)

        wait_gather(1, j0 + 1)
        mul_rows(1, j0 + 1)
        issue_scatter(1, j0 + 1)

        wait_scatter(0, j0)
        issue_gather(0, j0 + 3)

        wait_gather(2, j0 + 2)
        mul_rows(2, j0 + 2)
        issue_scatter(2, j0 + 2)

        wait_scatter(1, j0 + 1)

        @pl.when(j0 < _WIN - 4)
        def _():
          issue_gather(1, j0 + 4)

      wait_gather(0, _WIN - 1)
      mul_rows(0, _WIN - 1)
      issue_scatter(0, _WIN - 1)
      wait_scatter(2, _WIN - 2)
      wait_scatter(0, _WIN - 1)

    plsc.subcore_barrier()
    pltpu.sync_copy(acc.at[pl.ds(s * _RPS, _RPS)],
                    out_hbm.at[c, pl.ds(s * _RPS, _RPS)])

  return agg


_agg128 = _make_agg(_H)
_agg48 = _make_agg(_CP)


def _tc_a_body(degp_ref, x_ref, w1_ref, hp_ref, dinv_ref):
  deg = degp_ref[:, 0:1] + degp_ref[:, 1:2] + 1.0
  dinv = lax.rsqrt(deg)
  p = jnp.dot(x_ref[...], w1_ref[...], preferred_element_type=jnp.float32)
  hp_ref[...] = p * dinv
  dinv_ref[...] = dinv


def _tc_a(degp_t, x, W1):
  return pl.pallas_call(
      _tc_a_body,
      grid=(_N // _RB,),
      in_specs=[
          pl.BlockSpec((_RB, _NC), lambda i: (i, 0)),
          pl.BlockSpec((_RB, _F), lambda i: (i, 0)),
          pl.BlockSpec((_F, _H), lambda i: (0, 0)),
      ],
      out_specs=[
          pl.BlockSpec((_RB, _H), lambda i: (i, 0)),
          pl.BlockSpec((_RB, 1), lambda i: (i, 0)),
      ],
      out_shape=[
          jax.ShapeDtypeStruct((_N, _H), jnp.float32),
          jax.ShapeDtypeStruct((_N, 1), jnp.float32),
      ],
  )(degp_t, x, W1)


def _tc_b_body(agg_ref, hp_ref, dinv_ref, b1_ref, w2_ref, h2p_ref):
  ssum = agg_ref[0] + agg_ref[1] + hp_ref[...]
  o1 = jnp.maximum(ssum * dinv_ref[...] + b1_ref[...], 0.0)
  p2 = jnp.dot(o1, w2_ref[...], preferred_element_type=jnp.float32)
  h2p = p2 * dinv_ref[...]
  h2p_ref[...] = jnp.concatenate(
      [h2p, jnp.zeros((_RB, _CP - _C), jnp.float32)], axis=1)


def _tc_b(agg1, hp, dinv, b1, W2):
  return pl.pallas_call(
      _tc_b_body,
      grid=(_N // _RB,),
      in_specs=[
          pl.BlockSpec((_NC, _RB, _H), lambda i: (0, i, 0)),
          pl.BlockSpec((_RB, _H), lambda i: (i, 0)),
          pl.BlockSpec((_RB, 1), lambda i: (i, 0)),
          pl.BlockSpec((1, _H), lambda i: (0, 0)),
          pl.BlockSpec((_H, _C), lambda i: (0, 0)),
      ],
      out_specs=pl.BlockSpec((_RB, _CP), lambda i: (i, 0)),
      out_shape=jax.ShapeDtypeStruct((_N, _CP), jnp.float32),
  )(agg1, hp, dinv, b1, W2)


def _tc_c_body(agg_ref, h2p_ref, dinv_ref, b2_ref, out_ref):
  ssum = (agg_ref[0] + agg_ref[1] + h2p_ref[...]) * dinv_ref[...]
  v = ssum[:, :_C] + b2_ref[...]
  m = jnp.max(v, axis=1, keepdims=True)
  lse = jnp.log(jnp.sum(jnp.exp(v - m), axis=1, keepdims=True)) + m
  out_ref[...] = v - lse


def _tc_c(agg2, h2p, dinv, b2):
  return pl.pallas_call(
      _tc_c_body,
      grid=(_N // _RB,),
      in_specs=[
          pl.BlockSpec((_NC, _RB, _CP), lambda i: (0, i, 0)),
          pl.BlockSpec((_RB, _CP), lambda i: (i, 0)),
          pl.BlockSpec((_RB, 1), lambda i: (i, 0)),
          pl.BlockSpec((1, _C), lambda i: (0, 0)),
      ],
      out_specs=pl.BlockSpec((_RB, _C), lambda i: (i, 0)),
      out_shape=jax.ShapeDtypeStruct((_N, _C), jnp.float32),
  )(agg2, h2p, dinv, b2)


def kernel(x, edge_index, edge_weight, W1, b1, W2, b2):
  src = edge_index[0].reshape(_NW, _NWIN, _WIN, _CH)
  dst = edge_index[1].reshape(_NW, _NWIN, _WIN, _CH)
  w4 = edge_weight.reshape(_NW, _NWIN, _WIN, _CH)

  degp = _deg_kernel(dst, w4)
  degp_t = degp.T[:_N]
  hp, dinv = _tc_a(degp_t, x, W1)
  agg1 = _agg128(hp, src, dst, w4)
  h2p = _tc_b(agg1, hp, dinv, b1.reshape(1, _H), W2)
  agg2 = _agg48(h2p, src, dst, w4)
  return _tc_c(agg2, h2p, dinv, b2.reshape(1, _C))


# speedup vs baseline: 2.0674x; 1.0060x over previous
import functools

import jax
import jax.numpy as jnp
from jax import lax
from jax.experimental import pallas as pl
from jax.experimental.pallas import tpu as pltpu
from jax.experimental.pallas import tpu_sc as plsc

_N = 10000
_NPAD = 10240
_F = 128
_H = 128
_C = 40
_CP = 48
_E = 320000
_NC = 2
_NS = 16
_NW = _NC * _NS
_EPW = _E // _NW
_CH = 80
_NCHUNK = _EPW // _CH
_WIN = 25
_NWIN = _NCHUNK // _WIN
_RPS = _NPAD // _NS
_RB = 1000

_mesh = plsc.VectorSubcoreMesh(core_axis_name="c", subcore_axis_name="s")


@functools.partial(
    pl.kernel,
    out_type=jax.ShapeDtypeStruct((_NC, _NPAD), jnp.float32),
    mesh=_mesh,
    scratch_types=[
        pltpu.VMEM_SHARED((_NPAD,), jnp.float32),
        pltpu.VMEM((_NWIN, _WIN, _CH), jnp.int32),
        pltpu.VMEM((_NWIN, _WIN, _CH), jnp.float32),
        pltpu.VMEM((_RPS,), jnp.float32),
    ],
    compiler_params=pltpu.CompilerParams(use_tc_tiling_on_sc=False),
)
def _deg_kernel(dst_hbm, w_hbm, out_hbm, acc, di, wi, zbuf):
  c = lax.axis_index("c")
  s = lax.axis_index("s")
  wid = c * _NS + s

  @pl.loop(0, _RPS, step=16)
  def _(r):
    zbuf[pl.ds(r, 16)] = jnp.zeros((16,), jnp.float32)

  pltpu.sync_copy(zbuf, acc.at[pl.ds(s * _RPS, _RPS)])
  pltpu.sync_copy(dst_hbm.at[wid], di)
  pltpu.sync_copy(w_hbm.at[wid], wi)
  plsc.subcore_barrier()

  @pl.loop(0, _NWIN)
  def _(jw):
    @pl.loop(0, _WIN)
    def _(j):
      pltpu.sync_copy(wi.at[jw, j], acc.at[di.at[jw, j]], add=True)

  plsc.subcore_barrier()
  pltpu.sync_copy(acc.at[pl.ds(s * _RPS, _RPS)],
                  out_hbm.at[c, pl.ds(s * _RPS, _RPS)])


def _make_agg(D):
  @functools.partial(
      pl.kernel,
      out_type=jax.ShapeDtypeStruct((_NC, _NPAD, D), jnp.float32),
      mesh=_mesh,
      scratch_types=[
          pltpu.VMEM_SHARED((_NPAD, D), jnp.float32),
          pltpu.VMEM((_WIN, _CH), jnp.int32),
          pltpu.VMEM((_WIN, _CH), jnp.int32),
          pltpu.VMEM((_WIN, _CH), jnp.float32),
          pltpu.VMEM((3, _CH, D), jnp.float32),
          pltpu.SemaphoreType.DMA((3,)),
          pltpu.SemaphoreType.DMA((3,)),
      ],
      compiler_params=pltpu.CompilerParams(use_tc_tiling_on_sc=False),
  )
  def agg(h_hbm, src_hbm, dst_hbm, w_hbm, out_hbm, acc, si, di, wi, rows,
          gsem, ssem):
    c = lax.axis_index("c")
    s = lax.axis_index("s")
    wid = c * _NS + s

    @pl.loop(0, _CH)
    def _(r):
      for d in range(D // 16):
        rows[0, r, pl.ds(d * 16, 16)] = jnp.zeros((16,), jnp.float32)

    @pl.loop(0, _RPS, step=_CH)
    def _(r0):
      pltpu.sync_copy(rows.at[0], acc.at[pl.ds(s * _RPS + r0, _CH)])

    plsc.subcore_barrier()

    @pl.loop(0, _NWIN)
    def _(jw):
      pltpu.sync_copy(src_hbm.at[wid, jw], si)
      pltpu.sync_copy(dst_hbm.at[wid, jw], di)
      pltpu.sync_copy(w_hbm.at[wid, jw], wi)

      def mul_rows(p, j):
        @plsc.parallel_loop(0, _CH, step=16)
        def _(g):
          wvec = wi[j, pl.ds(g, 16)]
          for l in range(16):
            wv = wvec[l]
            for d in range(D // 16):
              rows[p, g + l, pl.ds(d * 16, 16)] = (
                  rows[p, g + l, pl.ds(d * 16, 16)] * wv)

      def wait_gather(p, j):
        pltpu.make_async_copy(h_hbm.at[si.at[j]], rows.at[p],
                              gsem.at[p]).wait()

      def issue_gather(p, j):
        pltpu.async_copy(h_hbm.at[si.at[j]], rows.at[p], gsem.at[p])

      def issue_scatter(p, j):
        pltpu.async_copy(rows.at[p], acc.at[di.at[j]], ssem.at[p], add=True)

      def wait_scatter(p, j):
        pltpu.make_async_copy(rows.at[p], acc.at[di.at[j]],
                              ssem.at[p]).wait()

      issue_gather(0, 0)
      issue_gather(1, 1)

      @pl.loop(0, _WIN - 1, step=3)
      def _(j0):
        wait_gather(0, j0)
        mul_rows(0, j0)
        issue_scatter(0, j0)

        @pl.when(j0 > 0)
        def _():
          wait_scatter(2, j0 - 1)
        issue_gather(2, j0 + 2)

        wait_gather(1, j0 + 1)
        mul_rows(1, j0 + 1)
        issue_scatter(1, j0 + 1)

        wait_scatter(0, j0)
        issue_gather(0, j0 + 3)

        wait_gather(2, j0 + 2)
        mul_rows(2, j0 + 2)
        issue_scatter(2, j0 + 2)

        wait_scatter(1, j0 + 1)

        @pl.when(j0 < _WIN - 4)
        def _():
          issue_gather(1, j0 + 4)

      wait_gather(0, _WIN - 1)
      mul_rows(0, _WIN - 1)
      issue_scatter(0, _WIN - 1)
      wait_scatter(2, _WIN - 2)
      wait_scatter(0, _WIN - 1)

    plsc.subcore_barrier()
    pltpu.sync_copy(acc.at[pl.ds(s * _RPS, _RPS)],
                    out_hbm.at[c, pl.ds(s * _RPS, _RPS)])

  return agg


_agg128 = _make_agg(_H)
_agg48 = _make_agg(_CP)


def _tc_a_body(degp_ref, x_ref, w1_ref, hp_ref, dinv_ref):
  deg = degp_ref[:, 0:1] + degp_ref[:, 1:2] + 1.0
  dinv = lax.rsqrt(deg)
  p = jnp.dot(x_ref[...], w1_ref[...], preferred_element_type=jnp.float32)
  hp_ref[...] = p * dinv
  dinv_ref[...] = dinv


def _tc_a(degp_t, x, W1):
  return pl.pallas_call(
      _tc_a_body,
      grid=(_N // _RB,),
      in_specs=[
          pl.BlockSpec((_RB, _NC), lambda i: (i, 0)),
          pl.BlockSpec((_RB, _F), lambda i: (i, 0)),
          pl.BlockSpec((_F, _H), lambda i: (0, 0)),
      ],
      out_specs=[
          pl.BlockSpec((_RB, _H), lambda i: (i, 0)),
          pl.BlockSpec((_RB, 1), lambda i: (i, 0)),
      ],
      out_shape=[
          jax.ShapeDtypeStruct((_N, _H), jnp.float32),
          jax.ShapeDtypeStruct((_N, 1), jnp.float32),
      ],
  )(degp_t, x, W1)


def _tc_b_body(agg_ref, hp_ref, dinv_ref, b1_ref, w2_ref, h2p_ref):
  ssum = agg_ref[0] + agg_ref[1] + hp_ref[...]
  o1 = jnp.maximum(ssum * dinv_ref[...] + b1_ref[...], 0.0)
  p2 = jnp.dot(o1, w2_ref[...], preferred_element_type=jnp.float32)
  h2p = p2 * dinv_ref[...]
  h2p_ref[...] = jnp.concatenate(
      [h2p, jnp.zeros((_RB, _CP - _C), jnp.float32)], axis=1)


def _tc_b(agg1, hp, dinv, b1, W2):
  return pl.pallas_call(
      _tc_b_body,
      grid=(_N // _RB,),
      in_specs=[
          pl.BlockSpec((_NC, _RB, _H), lambda i: (0, i, 0)),
          pl.BlockSpec((_RB, _H), lambda i: (i, 0)),
          pl.BlockSpec((_RB, 1), lambda i: (i, 0)),
          pl.BlockSpec((1, _H), lambda i: (0, 0)),
          pl.BlockSpec((_H, _C), lambda i: (0, 0)),
      ],
      out_specs=pl.BlockSpec((_RB, _CP), lambda i: (i, 0)),
      out_shape=jax.ShapeDtypeStruct((_N, _CP), jnp.float32),
  )(agg1, hp, dinv, b1, W2)


def _tc_c_body(agg_ref, h2p_ref, dinv_ref, b2_ref, out_ref):
  ssum = (agg_ref[0] + agg_ref[1] + h2p_ref[...]) * dinv_ref[...]
  v = ssum[:, :_C] + b2_ref[...]
  m = jnp.max(v, axis=1, keepdims=True)
  lse = jnp.log(jnp.sum(jnp.exp(v - m), axis=1, keepdims=True)) + m
  out_ref[...] = v - lse


def _tc_c(agg2, h2p, dinv, b2):
  return pl.pallas_call(
      _tc_c_body,
      grid=(_N // _RB,),
      in_specs=[
          pl.BlockSpec((_NC, _RB, _CP), lambda i: (0, i, 0)),
          pl.BlockSpec((_RB, _CP), lambda i: (i, 0)),
          pl.BlockSpec((_RB, 1), lambda i: (i, 0)),
          pl.BlockSpec((1, _C), lambda i: (0, 0)),
      ],
      out_specs=pl.BlockSpec((_RB, _C), lambda i: (i, 0)),
      out_shape=jax.ShapeDtypeStruct((_N, _C), jnp.float32),
  )(agg2, h2p, dinv, b2)


def kernel(x, edge_index, edge_weight, W1, b1, W2, b2):
  src = edge_index[0].reshape(_NW, _NWIN, _WIN, _CH)
  dst = edge_index[1].reshape(_NW, _NWIN, _WIN, _CH)
  w4 = edge_weight.reshape(_NW, _NWIN, _WIN, _CH)

  degp = _deg_kernel(dst, w4)
  degp_t = degp.T[:_N]
  hp, dinv = _tc_a(degp_t, x, W1)
  agg1 = _agg128(hp, src, dst, w4)
  h2p = _tc_b(agg1, hp, dinv, b1.reshape(1, _H), W2)
  agg2 = _agg48(h2p, src, dst, w4)
  return _tc_c(agg2, h2p, dinv, b2.reshape(1, _C))


# speedup vs baseline: 2.1797x; 1.0543x over previous
import functools

import jax
import jax.numpy as jnp
from jax import lax
from jax.experimental import pallas as pl
from jax.experimental.pallas import tpu as pltpu
from jax.experimental.pallas import tpu_sc as plsc

_N = 10000
_NPAD = 10240
_F = 128
_H = 128
_C = 40
_CP = 48
_E = 320000
_NC = 2
_NS = 16
_NW = _NC * _NS
_EPW = _E // _NW
_CH = 80
_NCHUNK = _EPW // _CH
_WIN = 25
_NWIN = _NCHUNK // _WIN
_RPS = _NPAD // _NS
_RB = 2000

_mesh = plsc.VectorSubcoreMesh(core_axis_name="c", subcore_axis_name="s")


@functools.partial(
    pl.kernel,
    out_type=jax.ShapeDtypeStruct((_NC, _NPAD), jnp.float32),
    mesh=_mesh,
    scratch_types=[
        pltpu.VMEM_SHARED((_NPAD,), jnp.float32),
        pltpu.VMEM((_NWIN, _WIN, _CH), jnp.int32),
        pltpu.VMEM((_NWIN, _WIN, _CH), jnp.float32),
        pltpu.VMEM((_RPS,), jnp.float32),
    ],
    compiler_params=pltpu.CompilerParams(use_tc_tiling_on_sc=False),
)
def _deg_kernel(ei_hbm, w_hbm, out_hbm, acc, di, wi, zbuf):
  c = lax.axis_index("c")
  s = lax.axis_index("s")
  wid = c * _NS + s

  @pl.loop(0, _RPS, step=16)
  def _(r):
    zbuf[pl.ds(r, 16)] = jnp.zeros((16,), jnp.float32)

  pltpu.sync_copy(zbuf, acc.at[pl.ds(s * _RPS, _RPS)])
  pltpu.sync_copy(ei_hbm.at[1, wid], di)
  pltpu.sync_copy(w_hbm.at[wid], wi)
  plsc.subcore_barrier()

  @pl.loop(0, _NWIN)
  def _(jw):
    @pl.loop(0, _WIN)
    def _(j):
      pltpu.sync_copy(wi.at[jw, j], acc.at[di.at[jw, j]], add=True)

  plsc.subcore_barrier()
  pltpu.sync_copy(acc.at[pl.ds(s * _RPS, _RPS)],
                  out_hbm.at[c, pl.ds(s * _RPS, _RPS)])


def _make_agg(D):
  @functools.partial(
      pl.kernel,
      out_type=jax.ShapeDtypeStruct((_NC, _NPAD, D), jnp.float32),
      mesh=_mesh,
      scratch_types=[
          pltpu.VMEM_SHARED((_NPAD, D), jnp.float32),
          pltpu.VMEM((_WIN, _CH), jnp.int32),
          pltpu.VMEM((_WIN, _CH), jnp.int32),
          pltpu.VMEM((_WIN, _CH), jnp.float32),
          pltpu.VMEM((3, _CH, D), jnp.float32),
          pltpu.SemaphoreType.DMA((3,)),
          pltpu.SemaphoreType.DMA((3,)),
      ],
      compiler_params=pltpu.CompilerParams(use_tc_tiling_on_sc=False),
  )
  def agg(h_hbm, ei_hbm, w_hbm, out_hbm, acc, si, di, wi, rows,
          gsem, ssem):
    c = lax.axis_index("c")
    s = lax.axis_index("s")
    wid = c * _NS + s

    @pl.loop(0, _CH)
    def _(r):
      for d in range(D // 16):
        rows[0, r, pl.ds(d * 16, 16)] = jnp.zeros((16,), jnp.float32)

    @pl.loop(0, _RPS, step=_CH)
    def _(r0):
      pltpu.sync_copy(rows.at[0], acc.at[pl.ds(s * _RPS + r0, _CH)])

    plsc.subcore_barrier()

    @pl.loop(0, _NWIN)
    def _(jw):
      pltpu.sync_copy(ei_hbm.at[0, wid, jw], si)
      pltpu.sync_copy(ei_hbm.at[1, wid, jw], di)
      pltpu.sync_copy(w_hbm.at[wid, jw], wi)

      def mul_rows(p, j):
        @plsc.parallel_loop(0, _CH, step=16)
        def _(g):
          wvec = wi[j, pl.ds(g, 16)]
          for l in range(16):
            wv = wvec[l]
            for d in range(D // 16):
              rows[p, g + l, pl.ds(d * 16, 16)] = (
                  rows[p, g + l, pl.ds(d * 16, 16)] * wv)

      def wait_gather(p, j):
        pltpu.make_async_copy(h_hbm.at[si.at[j]], rows.at[p],
                              gsem.at[p]).wait()

      def issue_gather(p, j):
        pltpu.async_copy(h_hbm.at[si.at[j]], rows.at[p], gsem.at[p])

      def issue_scatter(p, j):
        pltpu.async_copy(rows.at[p], acc.at[di.at[j]], ssem.at[p], add=True)

      def wait_scatter(p, j):
        pltpu.make_async_copy(rows.at[p], acc.at[di.at[j]],
                              ssem.at[p]).wait()

      issue_gather(0, 0)
      issue_gather(1, 1)

      @pl.loop(0, _WIN - 1, step=3)
      def _(j0):
        wait_gather(0, j0)
        mul_rows(0, j0)
        issue_scatter(0, j0)

        @pl.when(j0 > 0)
        def _():
          wait_scatter(2, j0 - 1)
        issue_gather(2, j0 + 2)

        wait_gather(1, j0 + 1)
        mul_rows(1, j0 + 1)
        issue_scatter(1, j0 + 1)

        wait_scatter(0, j0)
        issue_gather(0, j0 + 3)

        wait_gather(2, j0 + 2)
        mul_rows(2, j0 + 2)
        issue_scatter(2, j0 + 2)

        wait_scatter(1, j0 + 1)

        @pl.when(j0 < _WIN - 4)
        def _():
          issue_gather(1, j0 + 4)

      wait_gather(0, _WIN - 1)
      mul_rows(0, _WIN - 1)
      issue_scatter(0, _WIN - 1)
      wait_scatter(2, _WIN - 2)
      wait_scatter(0, _WIN - 1)

    plsc.subcore_barrier()
    pltpu.sync_copy(acc.at[pl.ds(s * _RPS, _RPS)],
                    out_hbm.at[c, pl.ds(s * _RPS, _RPS)])

  return agg


_agg128 = _make_agg(_H)
_agg48 = _make_agg(_CP)


def _tc_a_body(degp_ref, x_ref, w1_ref, hp_ref, dinv_ref):
  deg = degp_ref[:, 0:1] + degp_ref[:, 1:2] + 1.0
  dinv = lax.rsqrt(deg)
  p = jnp.dot(x_ref[...], w1_ref[...], preferred_element_type=jnp.float32)
  hp_ref[...] = p * dinv
  dinv_ref[...] = dinv


def _tc_a(degp_t, x, W1):
  return pl.pallas_call(
      _tc_a_body,
      grid=(_N // _RB,),
      in_specs=[
          pl.BlockSpec((_RB, _NC), lambda i: (i, 0)),
          pl.BlockSpec((_RB, _F), lambda i: (i, 0)),
          pl.BlockSpec((_F, _H), lambda i: (0, 0)),
      ],
      out_specs=[
          pl.BlockSpec((_RB, _H), lambda i: (i, 0)),
          pl.BlockSpec((_RB, 1), lambda i: (i, 0)),
      ],
      out_shape=[
          jax.ShapeDtypeStruct((_N, _H), jnp.float32),
          jax.ShapeDtypeStruct((_N, 1), jnp.float32),
      ],
  )(degp_t, x, W1)


def _tc_b_body(agg_ref, hp_ref, dinv_ref, b1_ref, w2_ref, h2p_ref):
  ssum = agg_ref[0] + agg_ref[1] + hp_ref[...]
  o1 = jnp.maximum(ssum * dinv_ref[...] + b1_ref[...], 0.0)
  p2 = jnp.dot(o1, w2_ref[...], preferred_element_type=jnp.float32)
  h2p = p2 * dinv_ref[...]
  h2p_ref[...] = jnp.concatenate(
      [h2p, jnp.zeros((_RB, _CP - _C), jnp.float32)], axis=1)


def _tc_b(agg1, hp, dinv, b1, W2):
  return pl.pallas_call(
      _tc_b_body,
      grid=(_N // _RB,),
      in_specs=[
          pl.BlockSpec((_NC, _RB, _H), lambda i: (0, i, 0)),
          pl.BlockSpec((_RB, _H), lambda i: (i, 0)),
          pl.BlockSpec((_RB, 1), lambda i: (i, 0)),
          pl.BlockSpec((1, _H), lambda i: (0, 0)),
          pl.BlockSpec((_H, _C), lambda i: (0, 0)),
      ],
      out_specs=pl.BlockSpec((_RB, _CP), lambda i: (i, 0)),
      out_shape=jax.ShapeDtypeStruct((_N, _CP), jnp.float32),
  )(agg1, hp, dinv, b1, W2)


def _tc_c_body(agg_ref, h2p_ref, dinv_ref, b2_ref, out_ref):
  ssum = (agg_ref[0] + agg_ref[1] + h2p_ref[...]) * dinv_ref[...]
  v = ssum[:, :_C] + b2_ref[...]
  m = jnp.max(v, axis=1, keepdims=True)
  lse = jnp.log(jnp.sum(jnp.exp(v - m), axis=1, keepdims=True)) + m
  out_ref[...] = v - lse


def _tc_c(agg2, h2p, dinv, b2):
  return pl.pallas_call(
      _tc_c_body,
      grid=(_N // _RB,),
      in_specs=[
          pl.BlockSpec((_NC, _RB, _CP), lambda i: (0, i, 0)),
          pl.BlockSpec((_RB, _CP), lambda i: (i, 0)),
          pl.BlockSpec((_RB, 1), lambda i: (i, 0)),
          pl.BlockSpec((1, _C), lambda i: (0, 0)),
      ],
      out_specs=pl.BlockSpec((_RB, _C), lambda i: (i, 0)),
      out_shape=jax.ShapeDtypeStruct((_N, _C), jnp.float32),
  )(agg2, h2p, dinv, b2)


def kernel(x, edge_index, edge_weight, W1, b1, W2, b2):
  ei5 = edge_index.reshape(2, _NW, _NWIN, _WIN, _CH)
  w4 = edge_weight.reshape(_NW, _NWIN, _WIN, _CH)

  degp = _deg_kernel(ei5, w4)
  degp_t = degp.T[:_N]
  hp, dinv = _tc_a(degp_t, x, W1)
  agg1 = _agg128(hp, ei5, w4)
  h2p = _tc_b(agg1, hp, dinv, b1.reshape(1, _H), W2)
  agg2 = _agg48(h2p, ei5, w4)
  return _tc_c(agg2, h2p, dinv, b2.reshape(1, _C))


# speedup vs baseline: 2.1837x; 1.0019x over previous
import functools

import jax
import jax.numpy as jnp
from jax import lax
from jax.experimental import pallas as pl
from jax.experimental.pallas import tpu as pltpu
from jax.experimental.pallas import tpu_sc as plsc

_N = 10000
_NPAD = 10240
_F = 128
_H = 128
_C = 40
_CP = 48
_E = 320000
_NC = 2
_NS = 16
_NW = _NC * _NS
_EPW = _E // _NW
_CH = 80
_NCHUNK = _EPW // _CH
_WIN = 25
_NWIN = _NCHUNK // _WIN
_RPS = _NPAD // _NS
_RB = 2000

_mesh = plsc.VectorSubcoreMesh(core_axis_name="c", subcore_axis_name="s")


@functools.partial(
    pl.kernel,
    out_type=jax.ShapeDtypeStruct((_NC, _NPAD), jnp.float32),
    mesh=_mesh,
    scratch_types=[
        pltpu.VMEM_SHARED((_NPAD,), jnp.float32),
        pltpu.VMEM((_NWIN, _WIN, _CH), jnp.int32),
        pltpu.VMEM((_NWIN, _WIN, _CH), jnp.float32),
        pltpu.VMEM((_RPS,), jnp.float32),
    ],
    compiler_params=pltpu.CompilerParams(use_tc_tiling_on_sc=False),
)
def _deg_kernel(ei_hbm, w_hbm, out_hbm, acc, di, wi, zbuf):
  c = lax.axis_index("c")
  s = lax.axis_index("s")
  wid = c * _NS + s

  @pl.loop(0, _RPS, step=16)
  def _(r):
    zbuf[pl.ds(r, 16)] = jnp.zeros((16,), jnp.float32)

  pltpu.sync_copy(zbuf, acc.at[pl.ds(s * _RPS, _RPS)])
  pltpu.sync_copy(ei_hbm.at[1, wid], di)
  pltpu.sync_copy(w_hbm.at[wid], wi)
  plsc.subcore_barrier()

  @pl.loop(0, _NWIN)
  def _(jw):
    @pl.loop(0, _WIN)
    def _(j):
      pltpu.sync_copy(wi.at[jw, j], acc.at[di.at[jw, j]], add=True)

  plsc.subcore_barrier()
  pltpu.sync_copy(acc.at[pl.ds(s * _RPS, _RPS)],
                  out_hbm.at[c, pl.ds(s * _RPS, _RPS)])


def _make_agg(D):
  @functools.partial(
      pl.kernel,
      out_type=jax.ShapeDtypeStruct((_NC, _NPAD, D), jnp.float32),
      mesh=_mesh,
      scratch_types=[
          pltpu.VMEM_SHARED((_NPAD, D), jnp.float32),
          pltpu.VMEM((_WIN, _CH), jnp.int32),
          pltpu.VMEM((_WIN, _CH), jnp.int32),
          pltpu.VMEM((_WIN, _CH), jnp.float32),
          pltpu.VMEM((3, _CH, D), jnp.float32),
          pltpu.SemaphoreType.DMA((3,)),
          pltpu.SemaphoreType.DMA((3,)),
      ],
      compiler_params=pltpu.CompilerParams(use_tc_tiling_on_sc=False),
  )
  def agg(h_hbm, ei_hbm, w_hbm, out_hbm, acc, si, di, wi, rows,
          gsem, ssem):
    c = lax.axis_index("c")
    s = lax.axis_index("s")
    wid = c * _NS + s

    @pl.loop(0, _CH)
    def _(r):
      for d in range(D // 16):
        rows[0, r, pl.ds(d * 16, 16)] = jnp.zeros((16,), jnp.float32)

    @pl.loop(0, _RPS, step=_CH)
    def _(r0):
      pltpu.sync_copy(rows.at[0], acc.at[pl.ds(s * _RPS + r0, _CH)])

    plsc.subcore_barrier()

    @pl.loop(0, _NWIN)
    def _(jw):
      pltpu.sync_copy(ei_hbm.at[0, wid, jw], si)
      pltpu.sync_copy(ei_hbm.at[1, wid, jw], di)
      pltpu.sync_copy(w_hbm.at[wid, jw], wi)

      def mul_rows(p, j):
        @plsc.parallel_loop(0, _CH, step=16)
        def _(g):
          wvec = wi[j, pl.ds(g, 16)]
          for l in range(16):
            wv = jnp.take_along_axis(wvec, jnp.full((16,), l, jnp.int32),
                                     axis=0)
            for d in range(D // 16):
              rows[p, g + l, pl.ds(d * 16, 16)] = (
                  rows[p, g + l, pl.ds(d * 16, 16)] * wv)

      def wait_gather(p, j):
        pltpu.make_async_copy(h_hbm.at[si.at[j]], rows.at[p],
                              gsem.at[p]).wait()

      def issue_gather(p, j):
        pltpu.async_copy(h_hbm.at[si.at[j]], rows.at[p], gsem.at[p])

      def issue_scatter(p, j):
        pltpu.async_copy(rows.at[p], acc.at[di.at[j]], ssem.at[p], add=True)

      def wait_scatter(p, j):
        pltpu.make_async_copy(rows.at[p], acc.at[di.at[j]],
                              ssem.at[p]).wait()

      issue_gather(0, 0)
      issue_gather(1, 1)

      @pl.loop(0, _WIN - 1, step=3)
      def _(j0):
        wait_gather(0, j0)
        mul_rows(0, j0)
        issue_scatter(0, j0)

        @pl.when(j0 > 0)
        def _():
          wait_scatter(2, j0 - 1)
        issue_gather(2, j0 + 2)

        wait_gather(1, j0 + 1)
        mul_rows(1, j0 + 1)
        issue_scatter(1, j0 + 1)

        wait_scatter(0, j0)
        issue_gather(0, j0 + 3)

        wait_gather(2, j0 + 2)
        mul_rows(2, j0 + 2)
        issue_scatter(2, j0 + 2)

        wait_scatter(1, j0 + 1)

        @pl.when(j0 < _WIN - 4)
        def _():
          issue_gather(1, j0 + 4)

      wait_gather(0, _WIN - 1)
      mul_rows(0, _WIN - 1)
      issue_scatter(0, _WIN - 1)
      wait_scatter(2, _WIN - 2)
      wait_scatter(0, _WIN - 1)

    plsc.subcore_barrier()
    pltpu.sync_copy(acc.at[pl.ds(s * _RPS, _RPS)],
                    out_hbm.at[c, pl.ds(s * _RPS, _RPS)])

  return agg


_agg128 = _make_agg(_H)
_agg48 = _make_agg(_CP)


def _tc_a_body(degp_ref, x_ref, w1_ref, hp_ref, dinv_ref):
  deg = degp_ref[:, 0:1] + degp_ref[:, 1:2] + 1.0
  dinv = lax.rsqrt(deg)
  p = jnp.dot(x_ref[...], w1_ref[...], preferred_element_type=jnp.float32)
  hp_ref[...] = p * dinv
  dinv_ref[...] = dinv


def _tc_a(degp_t, x, W1):
  return pl.pallas_call(
      _tc_a_body,
      grid=(_N // _RB,),
      in_specs=[
          pl.BlockSpec((_RB, _NC), lambda i: (i, 0)),
          pl.BlockSpec((_RB, _F), lambda i: (i, 0)),
          pl.BlockSpec((_F, _H), lambda i: (0, 0)),
      ],
      out_specs=[
          pl.BlockSpec((_RB, _H), lambda i: (i, 0)),
          pl.BlockSpec((_RB, 1), lambda i: (i, 0)),
      ],
      out_shape=[
          jax.ShapeDtypeStruct((_N, _H), jnp.float32),
          jax.ShapeDtypeStruct((_N, 1), jnp.float32),
      ],
  )(degp_t, x, W1)


def _tc_b_body(agg_ref, hp_ref, dinv_ref, b1_ref, w2_ref, h2p_ref):
  ssum = agg_ref[0] + agg_ref[1] + hp_ref[...]
  o1 = jnp.maximum(ssum * dinv_ref[...] + b1_ref[...], 0.0)
  p2 = jnp.dot(o1, w2_ref[...], preferred_element_type=jnp.float32)
  h2p = p2 * dinv_ref[...]
  h2p_ref[...] = jnp.concatenate(
      [h2p, jnp.zeros((_RB, _CP - _C), jnp.float32)], axis=1)


def _tc_b(agg1, hp, dinv, b1, W2):
  return pl.pallas_call(
      _tc_b_body,
      grid=(_N // _RB,),
      in_specs=[
          pl.BlockSpec((_NC, _RB, _H), lambda i: (0, i, 0)),
          pl.BlockSpec((_RB, _H), lambda i: (i, 0)),
          pl.BlockSpec((_RB, 1), lambda i: (i, 0)),
          pl.BlockSpec((1, _H), lambda i: (0, 0)),
          pl.BlockSpec((_H, _C), lambda i: (0, 0)),
      ],
      out_specs=pl.BlockSpec((_RB, _CP), lambda i: (i, 0)),
      out_shape=jax.ShapeDtypeStruct((_N, _CP), jnp.float32),
  )(agg1, hp, dinv, b1, W2)


def _tc_c_body(agg_ref, h2p_ref, dinv_ref, b2_ref, out_ref):
  ssum = (agg_ref[0] + agg_ref[1] + h2p_ref[...]) * dinv_ref[...]
  v = ssum[:, :_C] + b2_ref[...]
  m = jnp.max(v, axis=1, keepdims=True)
  lse = jnp.log(jnp.sum(jnp.exp(v - m), axis=1, keepdims=True)) + m
  out_ref[...] = v - lse


def _tc_c(agg2, h2p, dinv, b2):
  return pl.pallas_call(
      _tc_c_body,
      grid=(_N // _RB,),
      in_specs=[
          pl.BlockSpec((_NC, _RB, _CP), lambda i: (0, i, 0)),
          pl.BlockSpec((_RB, _CP), lambda i: (i, 0)),
          pl.BlockSpec((_RB, 1), lambda i: (i, 0)),
          pl.BlockSpec((1, _C), lambda i: (0, 0)),
      ],
      out_specs=pl.BlockSpec((_RB, _C), lambda i: (i, 0)),
      out_shape=jax.ShapeDtypeStruct((_N, _C), jnp.float32),
  )(agg2, h2p, dinv, b2)


def kernel(x, edge_index, edge_weight, W1, b1, W2, b2):
  ei5 = edge_index.reshape(2, _NW, _NWIN, _WIN, _CH)
  w4 = edge_weight.reshape(_NW, _NWIN, _WIN, _CH)

  degp = _deg_kernel(ei5, w4)
  degp_t = degp.T[:_N]
  hp, dinv = _tc_a(degp_t, x, W1)
  agg1 = _agg128(hp, ei5, w4)
  h2p = _tc_b(agg1, hp, dinv, b1.reshape(1, _H), W2)
  agg2 = _agg48(h2p, ei5, w4)
  return _tc_c(agg2, h2p, dinv, b2.reshape(1, _C))
